# Initial kernel scaffold; baseline (speedup 1.0000x reference)
import jax
import jax.numpy as jnp
from jax.experimental import pallas as pl


def kernel(inputs, token_table, pos_table):
    raise NotImplementedError("write your pallas kernel here")


# speedup vs baseline: 1.4254x; 1.4254x over previous
import functools

import jax
import jax.numpy as jnp
import numpy as np
from jax import lax
from jax.experimental import pallas as pl
from jax.experimental.pallas import tpu as pltpu
from jax.experimental.pallas import tpu_sc as plsc

SEQ = 200
EMB = 32
BATCH = 4096
NROWS = BATCH * SEQ
NW = 32
ROWS_PER_W = NROWS // NW
CHUNK = 1600
NCHUNK = ROWS_PER_W // CHUNK
SUBG = 100
NSUBG = CHUNK // SUBG
REP = CHUNK // SEQ
SCALE = float(np.sqrt(np.float32(EMB)))

_mesh = plsc.VectorSubcoreMesh(core_axis_name="c", subcore_axis_name="s")


@functools.partial(
    pl.kernel,
    out_type=jax.ShapeDtypeStruct((NROWS, EMB), jnp.float32),
    mesh=_mesh,
    compiler_params=pltpu.CompilerParams(use_tc_tiling_on_sc=False),
    scratch_types=[
        pltpu.VMEM((NSUBG, SUBG), jnp.int32),
        pltpu.VMEM((CHUNK, EMB), jnp.float32),
        pltpu.VMEM((SEQ, EMB), jnp.float32),
        pltpu.SemaphoreType.DMA,
    ],
)
def _sc_embed(idx_hbm, table_hbm, pos_hbm, out_hbm, idx_v, rows_v, pos_v, sem):
    wid = lax.axis_index("s") * 2 + lax.axis_index("c")
    pltpu.sync_copy(pos_hbm, pos_v)

    def chunk_body(c, carry):
        r0 = (wid * NCHUNK + c) * NSUBG
        rb = (wid * NCHUNK + c) * CHUNK
        pltpu.sync_copy(idx_hbm.at[pl.ds(r0, NSUBG)], idx_v)
        copies = []
        for j in range(NSUBG):
            copies.append(
                pltpu.async_copy(
                    table_hbm.at[idx_v.at[j]],
                    rows_v.at[pl.ds(j * SUBG, SUBG)],
                    sem,
                )
            )
        for cp in copies:
            cp.wait()

        def pos_body(s, carry2):
            p0 = pos_v[s, pl.ds(0, 16)]
            p1 = pos_v[s, pl.ds(16, 16)]
            for k in range(REP):
                r = s + SEQ * k
                rows_v[r, pl.ds(0, 16)] = rows_v[r, pl.ds(0, 16)] * SCALE + p0
                rows_v[r, pl.ds(16, 16)] = rows_v[r, pl.ds(16, 16)] * SCALE + p1
            return carry2

        lax.fori_loop(0, SEQ, pos_body, 0)
        pltpu.sync_copy(rows_v, out_hbm.at[pl.ds(rb, CHUNK)])
        return carry

    lax.fori_loop(0, NCHUNK, chunk_body, 0)


def kernel(inputs, token_table, pos_table):
    idx = inputs.reshape(-1).astype(jnp.int32).reshape(NROWS // SUBG, SUBG)
    out = _sc_embed(idx, token_table, pos_table)
    return out.reshape(BATCH, SEQ, EMB)
